# Initial kernel scaffold; baseline (speedup 1.0000x reference)
import jax
import jax.numpy as jnp
from jax.experimental import pallas as pl


def kernel(nfeat, edge_index, edge_attr, W_edge, W_ne, b_ne, W_self, b_self):
    raise NotImplementedError("write your pallas kernel here")


# speedup vs baseline: 1.6926x; 1.6926x over previous
import functools

import jax
import jax.numpy as jnp
from jax import lax
from jax.experimental import pallas as pl
from jax.experimental.pallas import tpu as pltpu
from jax.experimental.pallas import tpu_sc as plsc

N = 10000
E = 320000
D_IN = 128
D_EDGE = 16
D_OUT = 128
DH = D_OUT // 2

NPAD = 10240
NBLK = NPAD // 128
CHUNK = 128
NCHUNKS = E // CHUNK
NSUB = 16

_HI = lax.Precision.HIGHEST


def _p_body(nf_ref, wn_ref, p_ref):
    p_ref[...] = jnp.dot(nf_ref[...], wn_ref[0], precision=_HI,
                         preferred_element_type=jnp.float32)


def _q_body(ea_ref, we_ref, wn_ref, bn_ref, q_ref):
    wf_h = jnp.dot(we_ref[...], wn_ref[0], precision=_HI,
                   preferred_element_type=jnp.float32)
    q_ref[0] = jnp.dot(ea_ref[...], wf_h, precision=_HI,
                       preferred_element_type=jnp.float32) + bn_ref[0]


def _fin_body(sums_ref, cnts_ref, nf_ref, ws_ref, bs_ref, o_ref):
    s = sums_ref[...]
    tot = jnp.concatenate([s[0], s[1]], axis=1)
    cnt = cnts_ref[...][:, 0:1]
    neigh = tot / jnp.maximum(cnt, 1.0)
    o_ref[...] = neigh + jnp.dot(nf_ref[...], ws_ref[...], precision=_HI,
                                 preferred_element_type=jnp.float32) + bs_ref[...]


def _sc_edge_kernel(ei, p_hbm, q_hbm, sums_out, cnts_out,
                    idxb, pbuf, qbuf, obuf, acc_s, acc_c, sem_g, sem_q):
    cid = lax.axis_index("c")
    sid = lax.axis_index("s")

    def _zrow(r, _):
        for cc in range(DH // 16):
            qbuf[r, pl.ds(cc * 16, 16)] = jnp.zeros((16,), jnp.float32)
        obuf[r, pl.ds(0, 16)] = jnp.zeros((16,), jnp.float32)
        return 0
    lax.fori_loop(0, CHUNK, _zrow, 0)

    def _zblk(k, _):
        b = sid + NSUB * k
        pltpu.sync_copy(qbuf, acc_s.at[pl.ds(b * 128, 128)])
        pltpu.sync_copy(obuf, acc_c.at[pl.ds(b * 128, 128)])
        return 0
    lax.fori_loop(0, NBLK // NSUB, _zblk, 0)
    plsc.subcore_barrier()

    def _orow(r, _):
        obuf[r, pl.ds(0, 16)] = jnp.ones((16,), jnp.float32)
        return 0
    lax.fori_loop(0, CHUNK, _orow, 0)

    nmy = (NCHUNKS - sid + NSUB - 1) // NSUB

    def _chunk(j, _):
        ch = sid + j * NSUB
        base = ch * CHUNK
        pltpu.sync_copy(ei.at[0, pl.ds(base, CHUNK)], idxb.at[0])
        pltpu.sync_copy(ei.at[1, pl.ds(base, CHUNK)], idxb.at[1])
        off = cid * N
        for g in range(CHUNK // 16):
            s = pl.ds(g * 16, 16)
            idxb[0, s] = idxb[0, s] + off
        g_cp = pltpu.async_copy(p_hbm.at[idxb.at[0]], pbuf, sem_g)
        q_cp = pltpu.async_copy(q_hbm.at[cid, pl.ds(base, CHUNK)], qbuf, sem_q)
        g_cp.wait()
        q_cp.wait()

        def _crow(r, _):
            for cc in range(DH // 16):
                s = pl.ds(cc * 16, 16)
                pbuf[r, s] = jnp.maximum(pbuf[r, s] + qbuf[r, s], 0.0)
            return 0
        lax.fori_loop(0, CHUNK, _crow, 0)
        pltpu.sync_copy(pbuf, acc_s.at[idxb.at[1]], add=True)

        @pl.when(cid == 0)
        def _():
            pltpu.sync_copy(obuf, acc_c.at[idxb.at[1]], add=True)
        return 0
    lax.fori_loop(0, nmy, _chunk, 0)
    plsc.subcore_barrier()

    def _dblk(k, _):
        b = sid + NSUB * k
        pltpu.sync_copy(acc_s.at[pl.ds(b * 128, 128)],
                        sums_out.at[cid, pl.ds(b * 128, 128)])

        @pl.when(cid == 0)
        def _():
            pltpu.sync_copy(acc_c.at[pl.ds(b * 128, 128)],
                            cnts_out.at[pl.ds(b * 128, 128)])
        return 0
    lax.fori_loop(0, NBLK // NSUB, _dblk, 0)


def kernel(nfeat, edge_index, edge_attr, W_edge, W_ne, b_ne, W_self, b_self):
    f32 = jnp.float32
    b_self2 = b_self.reshape(1, D_OUT)
    wn_top_h = jnp.stack([W_ne[:D_IN, :DH], W_ne[:D_IN, DH:]])
    wn_bot_h = jnp.stack([W_ne[D_IN:, :DH], W_ne[D_IN:, DH:]])
    bn_h = jnp.stack([b_ne[:DH], b_ne[DH:]]).reshape(2, 1, DH)

    PB = 400
    P = pl.pallas_call(
        _p_body,
        grid=(2, N // PB),
        in_specs=[
            pl.BlockSpec((PB, D_IN), lambda h, i: (i, 0)),
            pl.BlockSpec((1, D_IN, DH), lambda h, i: (h, 0, 0)),
        ],
        out_specs=pl.BlockSpec((PB, DH), lambda h, i: (h * (N // PB) + i, 0)),
        out_shape=jax.ShapeDtypeStruct((2 * N, DH), f32),
    )(nfeat, wn_top_h)

    QB = 2000
    Q = pl.pallas_call(
        _q_body,
        grid=(2, E // QB),
        in_specs=[
            pl.BlockSpec((QB, D_EDGE), lambda h, i: (i, 0)),
            pl.BlockSpec((D_EDGE, D_OUT), lambda h, i: (0, 0)),
            pl.BlockSpec((1, D_OUT, DH), lambda h, i: (h, 0, 0)),
            pl.BlockSpec((1, 1, DH), lambda h, i: (h, 0, 0)),
        ],
        out_specs=pl.BlockSpec((1, QB, DH), lambda h, i: (h, i, 0)),
        out_shape=jax.ShapeDtypeStruct((2, E, DH), f32),
    )(edge_attr, W_edge, wn_bot_h, bn_h)

    mesh = plsc.VectorSubcoreMesh(core_axis_name="c", subcore_axis_name="s")
    sc = functools.partial(
        pl.kernel,
        mesh=mesh,
        compiler_params=pltpu.CompilerParams(use_tc_tiling_on_sc=False),
        out_type=(
            jax.ShapeDtypeStruct((2, NPAD, DH), f32),
            jax.ShapeDtypeStruct((NPAD, 16), f32),
        ),
        scratch_types=[
            pltpu.VMEM((2, CHUNK), jnp.int32),
            pltpu.VMEM((CHUNK, DH), f32),
            pltpu.VMEM((CHUNK, DH), f32),
            pltpu.VMEM((CHUNK, 16), f32),
            pltpu.VMEM_SHARED((NPAD, DH), f32),
            pltpu.VMEM_SHARED((NPAD, 16), f32),
            pltpu.SemaphoreType.DMA,
            pltpu.SemaphoreType.DMA,
        ],
    )(_sc_edge_kernel)
    sums, cnts = sc(edge_index, P, Q)

    FB = 400
    out = pl.pallas_call(
        _fin_body,
        grid=(N // FB,),
        in_specs=[
            pl.BlockSpec((2, FB, DH), lambda i: (0, i, 0)),
            pl.BlockSpec((FB, 16), lambda i: (i, 0)),
            pl.BlockSpec((FB, D_IN), lambda i: (i, 0)),
            pl.BlockSpec((D_IN, D_OUT), lambda i: (0, 0)),
            pl.BlockSpec((1, D_OUT), lambda i: (0, 0)),
        ],
        out_specs=pl.BlockSpec((FB, D_OUT), lambda i: (i, 0)),
        out_shape=jax.ShapeDtypeStruct((N, D_OUT), f32),
    )(sums, cnts, nfeat, W_self, b_self2)
    return out


# speedup vs baseline: 2.3361x; 1.3802x over previous
import functools

import jax
import jax.numpy as jnp
from jax import lax
from jax.experimental import pallas as pl
from jax.experimental.pallas import tpu as pltpu
from jax.experimental.pallas import tpu_sc as plsc

N = 10000
E = 320000
D_IN = 128
D_EDGE = 16
D_OUT = 128
DH = D_OUT // 2
PACK = 8
QW = PACK * DH

NPAD = 10240
NBLK = NPAD // 128
CHUNK = 128
QROWS = CHUNK // PACK
NCHUNKS = E // CHUNK
NSUB = 16


def _p_body(nf_ref, wn_ref, p_ref):
    p_ref[0] = jnp.dot(nf_ref[...], wn_ref[0],
                       preferred_element_type=jnp.float32)


def _q_body(ea_ref, we_ref, wn_ref, bn_ref, q_ref):
    wf = jnp.dot(we_ref[...], wn_ref[0],
                 preferred_element_type=jnp.float32)
    z = jnp.zeros((D_EDGE, DH), jnp.float32)
    rows = []
    for j in range(PACK):
        pieces = [wf if t == j else z for t in range(PACK)]
        rows.append(jnp.concatenate(pieces, axis=1))
    wbig = jnp.concatenate(rows, axis=0)
    bias = jnp.concatenate([bn_ref[0]] * PACK, axis=1)
    q_ref[0] = jnp.dot(ea_ref[...], wbig,
                       preferred_element_type=jnp.float32) + bias


def _fin_body(sums_ref, cnts_ref, nf_ref, ws_ref, bs_ref, o_ref):
    s = sums_ref[...]
    tot = jnp.concatenate([s[0], s[1]], axis=1)
    cnt = cnts_ref[...][:, 0:1]
    neigh = tot / jnp.maximum(cnt, 1.0)
    o_ref[...] = neigh + jnp.dot(nf_ref[...], ws_ref[...],
                                 preferred_element_type=jnp.float32) + bs_ref[...]


def _sc_edge_kernel(ei, p_hbm, q_hbm, sums_out, cnts_out,
                    idx0, idx1, pb0, pb1, qb0, qb1, obuf, p_spm, acc_s, acc_c,
                    sg0, sg1, sq0, sq1, ss0, ss1, sc0, sc1):
    cid = lax.axis_index("c")
    sid = lax.axis_index("s")

    idxs = (idx0, idx1)
    pbs = (pb0, pb1)
    qbs = (qb0, qb1)
    sgs = (sg0, sg1)
    sqs = (sq0, sq1)
    sss = (ss0, ss1)
    scs = (sc0, sc1)

    def _zrow(r, _):
        for cc in range(DH // 16):
            pb0[r, pl.ds(cc * 16, 16)] = jnp.zeros((16,), jnp.float32)
        obuf[r, pl.ds(0, 16)] = jnp.zeros((16,), jnp.float32)
        return 0
    lax.fori_loop(0, CHUNK, _zrow, 0)

    def _zblk(k, _):
        b = sid + NSUB * k
        pltpu.sync_copy(pb0, acc_s.at[pl.ds(b * 128, 128)])
        pltpu.sync_copy(obuf, acc_c.at[pl.ds(b * 128, 128)])
        pltpu.sync_copy(p_hbm.at[cid, pl.ds(b * 128, 128)],
                        p_spm.at[pl.ds(b * 128, 128)])
        return 0
    lax.fori_loop(0, NBLK // NSUB, _zblk, 0)
    plsc.subcore_barrier()

    def _orow(r, _):
        obuf[r, pl.ds(0, 16)] = jnp.ones((16,), jnp.float32)
        return 0
    lax.fori_loop(0, CHUNK, _orow, 0)

    def _load_idx(s, ch):
        base = ch * CHUNK
        pltpu.sync_copy(ei.at[0, pl.ds(base, CHUNK)], idxs[s].at[0])
        pltpu.sync_copy(ei.at[1, pl.ds(base, CHUNK)], idxs[s].at[1])

    def _start_fetch(s, ch):
        pltpu.async_copy(p_spm.at[idxs[s].at[0]], pbs[s], sgs[s])
        pltpu.async_copy(q_hbm.at[cid, pl.ds(ch * QROWS, QROWS)], qbs[s], sqs[s])

    def _wait_fetch(s):
        pltpu.make_async_copy(p_spm.at[idxs[s].at[0]], pbs[s], sgs[s]).wait()
        pltpu.make_async_copy(q_hbm.at[cid, pl.ds(0, QROWS)], qbs[s], sqs[s]).wait()

    def _start_scatter(s):
        pltpu.async_copy(pbs[s], acc_s.at[idxs[s].at[1]], sss[s], add=True)

        @pl.when(cid == 0)
        def _():
            pltpu.async_copy(obuf, acc_c.at[idxs[s].at[1]], scs[s], add=True)

    def _wait_scatter(s):
        pltpu.make_async_copy(pbs[s], acc_s.at[idxs[s].at[1]], sss[s]).wait()

        @pl.when(cid == 0)
        def _():
            pltpu.make_async_copy(obuf, acc_c.at[idxs[s].at[1]], scs[s]).wait()

    def _compute(s):
        pb, qb = pbs[s], qbs[s]

        def _crow(r, _):
            qrow = r >> 3
            qcol = (r & 7) * DH
            for cc in range(DH // 16):
                g = pl.ds(cc * 16, 16)
                pb[r, g] = jnp.maximum(
                    pb[r, g] + qb[qrow, pl.ds(qcol + cc * 16, 16)], 0.0)
            return 0
        lax.fori_loop(0, CHUNK, _crow, 0)

    nmy = (NCHUNKS - sid + NSUB - 1) // NSUB
    npairs = (nmy + 1) // 2

    _load_idx(0, sid)
    _start_fetch(0, sid)

    def _pair(k, _):
        for s in (0, 1):
            j = 2 * k + s
            ch = sid + j * NSUB
            ch_n = ch + NSUB
            nxt = 1 - s

            @pl.when(jnp.logical_and(ch < NCHUNKS, ch_n < NCHUNKS))
            def _():
                @pl.when(j >= 1)
                def _():
                    _wait_scatter(nxt)
                _load_idx(nxt, ch_n)
                _start_fetch(nxt, ch_n)

            @pl.when(ch < NCHUNKS)
            def _():
                _wait_fetch(s)
                _compute(s)
                _start_scatter(s)
        return 0
    lax.fori_loop(0, npairs, _pair, 0)

    _wait_scatter(0)
    _wait_scatter(1)
    plsc.subcore_barrier()

    def _dblk(k, _):
        b = sid + NSUB * k
        pltpu.sync_copy(acc_s.at[pl.ds(b * 128, 128)],
                        sums_out.at[cid, pl.ds(b * 128, 128)])

        @pl.when(cid == 0)
        def _():
            pltpu.sync_copy(acc_c.at[pl.ds(b * 128, 128)],
                            cnts_out.at[pl.ds(b * 128, 128)])
        return 0
    lax.fori_loop(0, NBLK // NSUB, _dblk, 0)


def kernel(nfeat, edge_index, edge_attr, W_edge, W_ne, b_ne, W_self, b_self):
    f32 = jnp.float32
    b_self2 = b_self.reshape(1, D_OUT)
    wn_top_h = jnp.stack([W_ne[:D_IN, :DH], W_ne[:D_IN, DH:]])
    wn_bot_h = jnp.stack([W_ne[D_IN:, :DH], W_ne[D_IN:, DH:]])
    bn_h = jnp.stack([b_ne[:DH], b_ne[DH:]]).reshape(2, 1, DH)
    ea8 = edge_attr.reshape(E // PACK, PACK * D_EDGE)

    PB = 400
    P = pl.pallas_call(
        _p_body,
        grid=(2, N // PB),
        in_specs=[
            pl.BlockSpec((PB, D_IN), lambda h, i: (i, 0)),
            pl.BlockSpec((1, D_IN, DH), lambda h, i: (h, 0, 0)),
        ],
        out_specs=pl.BlockSpec((1, PB, DH), lambda h, i: (h, i, 0)),
        out_shape=jax.ShapeDtypeStruct((2, NPAD, DH), f32),
    )(nfeat, wn_top_h)

    QB = 2000
    Q = pl.pallas_call(
        _q_body,
        grid=(2, E // PACK // QB),
        in_specs=[
            pl.BlockSpec((QB, PACK * D_EDGE), lambda h, i: (i, 0)),
            pl.BlockSpec((D_EDGE, D_OUT), lambda h, i: (0, 0)),
            pl.BlockSpec((1, D_OUT, DH), lambda h, i: (h, 0, 0)),
            pl.BlockSpec((1, 1, DH), lambda h, i: (h, 0, 0)),
        ],
        out_specs=pl.BlockSpec((1, QB, QW), lambda h, i: (h, i, 0)),
        out_shape=jax.ShapeDtypeStruct((2, E // PACK, QW), f32),
    )(ea8, W_edge, wn_bot_h, bn_h)

    mesh = plsc.VectorSubcoreMesh(core_axis_name="c", subcore_axis_name="s")
    sc = functools.partial(
        pl.kernel,
        mesh=mesh,
        compiler_params=pltpu.CompilerParams(use_tc_tiling_on_sc=False),
        out_type=(
            jax.ShapeDtypeStruct((2, NPAD, DH), f32),
            jax.ShapeDtypeStruct((NPAD, 16), f32),
        ),
        scratch_types=[
            pltpu.VMEM((2, CHUNK), jnp.int32),
            pltpu.VMEM((2, CHUNK), jnp.int32),
            pltpu.VMEM((CHUNK, DH), f32),
            pltpu.VMEM((CHUNK, DH), f32),
            pltpu.VMEM((QROWS, QW), f32),
            pltpu.VMEM((QROWS, QW), f32),
            pltpu.VMEM((CHUNK, 16), f32),
            pltpu.VMEM_SHARED((NPAD, DH), f32),
            pltpu.VMEM_SHARED((NPAD, DH), f32),
            pltpu.VMEM_SHARED((NPAD, 16), f32),
            pltpu.SemaphoreType.DMA,
            pltpu.SemaphoreType.DMA,
            pltpu.SemaphoreType.DMA,
            pltpu.SemaphoreType.DMA,
            pltpu.SemaphoreType.DMA,
            pltpu.SemaphoreType.DMA,
            pltpu.SemaphoreType.DMA,
            pltpu.SemaphoreType.DMA,
        ],
    )(_sc_edge_kernel)
    sums, cnts = sc(edge_index, P, Q)

    FB = 400
    out = pl.pallas_call(
        _fin_body,
        grid=(N // FB,),
        in_specs=[
            pl.BlockSpec((2, FB, DH), lambda i: (0, i, 0)),
            pl.BlockSpec((FB, 16), lambda i: (i, 0)),
            pl.BlockSpec((FB, D_IN), lambda i: (i, 0)),
            pl.BlockSpec((D_IN, D_OUT), lambda i: (0, 0)),
            pl.BlockSpec((1, D_OUT), lambda i: (0, 0)),
        ],
        out_specs=pl.BlockSpec((FB, D_OUT), lambda i: (i, 0)),
        out_shape=jax.ShapeDtypeStruct((N, D_OUT), f32),
    )(sums, cnts, nfeat, W_self, b_self2)
    return out


# speedup vs baseline: 3.3556x; 1.4364x over previous
import functools

import jax
import jax.numpy as jnp
from jax import lax
from jax.experimental import pallas as pl
from jax.experimental.pallas import tpu as pltpu
from jax.experimental.pallas import tpu_sc as plsc

N = 10000
E = 320000
D_IN = 128
D_EDGE = 16
D_OUT = 128
DH = D_OUT // 2
PACK = 8
QW = PACK * DH

NPAD = 10240
NBLK = NPAD // 128
CHUNK = 128
QROWS = CHUNK // PACK
NCHUNKS = E // CHUNK
NSUB = 16


def _p_body(nf_ref, wn_ref, p_ref):
    p_ref[0] = jnp.dot(nf_ref[...], wn_ref[0],
                       preferred_element_type=jnp.float32)


def _q_body(ea_ref, we_ref, wn_ref, bn_ref, q_ref):
    wf = jnp.dot(we_ref[...], wn_ref[0],
                 preferred_element_type=jnp.float32)
    z = jnp.zeros((D_EDGE, DH), jnp.float32)
    rows = []
    for j in range(PACK):
        pieces = [wf if t == j else z for t in range(PACK)]
        rows.append(jnp.concatenate(pieces, axis=1))
    wbig = jnp.concatenate(rows, axis=0)
    bias = jnp.concatenate([bn_ref[0]] * PACK, axis=1)
    q_ref[0] = jnp.dot(ea_ref[...], wbig,
                       preferred_element_type=jnp.float32) + bias


def _fin_body(sums_ref, cnts_ref, nf_ref, ws_ref, bs_ref, o_ref):
    s = sums_ref[...]
    tot = jnp.concatenate([s[0], s[1]], axis=1)
    cnt = cnts_ref[...][:, 0:1]
    neigh = tot / jnp.maximum(cnt, 1.0)
    o_ref[...] = neigh + jnp.dot(nf_ref[...], ws_ref[...],
                                 preferred_element_type=jnp.float32) + bs_ref[...]


def _sc_edge_kernel(ei3, p_hbm, q_hbm, sums_out, cnts_out,
                    idx0, idx1, pb0, pb1, qb0, qb1, obuf, p_spm, acc_s, acc_c,
                    sg0, sg1, sq0, sq1, ss0, ss1, sc0, sc1):
    cid = lax.axis_index("c")
    sid = lax.axis_index("s")

    idxs = (idx0, idx1)
    pbs = (pb0, pb1)
    qbs = (qb0, qb1)
    sgs = (sg0, sg1)
    sqs = (sq0, sq1)
    sss = (ss0, ss1)
    scs = (sc0, sc1)

    nmy = (NCHUNKS - sid + NSUB - 1) // NSUB

    def _zrow(r, _):
        for cc in range(DH // 16):
            pb0[r, pl.ds(cc * 16, 16)] = jnp.zeros((16,), jnp.float32)
        obuf[r, pl.ds(0, 16)] = jnp.zeros((16,), jnp.float32)
        return 0
    lax.fori_loop(0, CHUNK, _zrow, 0)

    def _zblk(k, _):
        b = sid + NSUB * k
        pltpu.sync_copy(pb0, acc_s.at[pl.ds(b * 128, 128)])
        pltpu.sync_copy(obuf, acc_c.at[pl.ds(b * 128, 128)])
        pltpu.sync_copy(p_hbm.at[cid, pl.ds(b * 128, 128)],
                        p_spm.at[pl.ds(b * 128, 128)])
        return 0
    lax.fori_loop(0, NBLK // NSUB, _zblk, 0)
    plsc.subcore_barrier()

    def _orow(r, _):
        obuf[r, pl.ds(0, 16)] = jnp.ones((16,), jnp.float32)
        return 0
    lax.fori_loop(0, CHUNK, _orow, 0)

    def _load_idx(s, j):
        ch = sid + j * NSUB
        pltpu.sync_copy(ei3.at[0, ch], idxs[s].at[0])
        pltpu.sync_copy(ei3.at[1, ch], idxs[s].at[1])

    def _start_fetch(s, j):
        ch = sid + j * NSUB
        pltpu.async_copy(p_spm.at[idxs[s].at[0]], pbs[s], sgs[s])
        pltpu.async_copy(q_hbm.at[cid, pl.ds(ch * QROWS, QROWS)],
                         qbs[s], sqs[s])

    def _wait_fetch(s):
        pltpu.make_async_copy(p_spm.at[idxs[s].at[0]], pbs[s], sgs[s]).wait()
        pltpu.make_async_copy(q_hbm.at[cid, pl.ds(0, QROWS)], qbs[s], sqs[s]).wait()

    def _start_scatter(s, j):
        pltpu.async_copy(pbs[s], acc_s.at[idxs[s].at[1]], sss[s], add=True)

        @pl.when(cid == 0)
        def _():
            pltpu.async_copy(obuf, acc_c.at[idxs[s].at[1]], scs[s], add=True)

    def _wait_scatter(s):
        pltpu.make_async_copy(pbs[s], acc_s.at[idxs[s].at[1]], sss[s]).wait()

        @pl.when(cid == 0)
        def _():
            pltpu.make_async_copy(obuf, acc_c.at[idxs[s].at[1]], scs[s]).wait()

    def _compute(s):
        pb, qb = pbs[s], qbs[s]

        @plsc.parallel_loop(0, CHUNK, 1, unroll=8)
        def _crow(r):
            qrow = r >> 3
            qcol = (r & 7) * DH
            for cc in range(DH // 16):
                g = pl.ds(cc * 16, 16)
                pb[r, g] = jnp.maximum(
                    pb[r, g] + qb[qrow, pl.ds(qcol + cc * 16, 16)], 0.0)

    npairs = (nmy + 1) // 2

    _load_idx(0, 0)
    _start_fetch(0, 0)

    def _pair(k, _):
        for s in (0, 1):
            j = 2 * k + s
            nxt = 1 - s

            @pl.when(j + 1 < nmy)
            def _():
                @pl.when(j >= 1)
                def _():
                    _wait_scatter(nxt)
                _load_idx(nxt, j + 1)
                _start_fetch(nxt, j + 1)

            @pl.when(j < nmy)
            def _():
                _wait_fetch(s)
                _compute(s)
                _start_scatter(s, j)
        return 0
    lax.fori_loop(0, npairs, _pair, 0)

    _wait_scatter(0)
    _wait_scatter(1)
    plsc.subcore_barrier()

    def _dblk(k, _):
        b = sid + NSUB * k
        pltpu.sync_copy(acc_s.at[pl.ds(b * 128, 128)],
                        sums_out.at[cid, pl.ds(b * 128, 128)])

        @pl.when(cid == 0)
        def _():
            pltpu.sync_copy(acc_c.at[pl.ds(b * 128, 128)],
                            cnts_out.at[pl.ds(b * 128, 128)])
        return 0
    lax.fori_loop(0, NBLK // NSUB, _dblk, 0)


def kernel(nfeat, edge_index, edge_attr, W_edge, W_ne, b_ne, W_self, b_self):
    f32 = jnp.float32
    b_self2 = b_self.reshape(1, D_OUT)
    wn_top_h = jnp.stack([W_ne[:D_IN, :DH], W_ne[:D_IN, DH:]])
    wn_bot_h = jnp.stack([W_ne[D_IN:, :DH], W_ne[D_IN:, DH:]])
    bn_h = jnp.stack([b_ne[:DH], b_ne[DH:]]).reshape(2, 1, DH)
    ea8 = edge_attr.reshape(E // PACK, PACK * D_EDGE)
    ei3 = edge_index.reshape(2, NCHUNKS, CHUNK)

    PB = 400
    P = pl.pallas_call(
        _p_body,
        grid=(2, N // PB),
        in_specs=[
            pl.BlockSpec((PB, D_IN), lambda h, i: (i, 0)),
            pl.BlockSpec((1, D_IN, DH), lambda h, i: (h, 0, 0)),
        ],
        out_specs=pl.BlockSpec((1, PB, DH), lambda h, i: (h, i, 0)),
        out_shape=jax.ShapeDtypeStruct((2, NPAD, DH), f32),
    )(nfeat, wn_top_h)

    QB = 2000
    Q = pl.pallas_call(
        _q_body,
        grid=(2, E // PACK // QB),
        in_specs=[
            pl.BlockSpec((QB, PACK * D_EDGE), lambda h, i: (i, 0)),
            pl.BlockSpec((D_EDGE, D_OUT), lambda h, i: (0, 0)),
            pl.BlockSpec((1, D_OUT, DH), lambda h, i: (h, 0, 0)),
            pl.BlockSpec((1, 1, DH), lambda h, i: (h, 0, 0)),
        ],
        out_specs=pl.BlockSpec((1, QB, QW), lambda h, i: (h, i, 0)),
        out_shape=jax.ShapeDtypeStruct((2, E // PACK, QW), f32),
    )(ea8, W_edge, wn_bot_h, bn_h)

    mesh = plsc.VectorSubcoreMesh(core_axis_name="c", subcore_axis_name="s")
    sc = functools.partial(
        pl.kernel,
        mesh=mesh,
        compiler_params=pltpu.CompilerParams(use_tc_tiling_on_sc=False),
        out_type=(
            jax.ShapeDtypeStruct((2, NPAD, DH), f32),
            jax.ShapeDtypeStruct((NPAD, 16), f32),
        ),
        scratch_types=[
            pltpu.VMEM((2, CHUNK), jnp.int32),
            pltpu.VMEM((2, CHUNK), jnp.int32),
            pltpu.VMEM((CHUNK, DH), f32),
            pltpu.VMEM((CHUNK, DH), f32),
            pltpu.VMEM((QROWS, QW), f32),
            pltpu.VMEM((QROWS, QW), f32),
            pltpu.VMEM((CHUNK, 16), f32),
            pltpu.VMEM_SHARED((NPAD, DH), f32),
            pltpu.VMEM_SHARED((NPAD, DH), f32),
            pltpu.VMEM_SHARED((NPAD, 16), f32),
            pltpu.SemaphoreType.DMA,
            pltpu.SemaphoreType.DMA,
            pltpu.SemaphoreType.DMA,
            pltpu.SemaphoreType.DMA,
            pltpu.SemaphoreType.DMA,
            pltpu.SemaphoreType.DMA,
            pltpu.SemaphoreType.DMA,
            pltpu.SemaphoreType.DMA,
        ],
    )(_sc_edge_kernel)
    sums, cnts = sc(ei3, P, Q)

    FB = 400
    out = pl.pallas_call(
        _fin_body,
        grid=(N // FB,),
        in_specs=[
            pl.BlockSpec((2, FB, DH), lambda i: (0, i, 0)),
            pl.BlockSpec((FB, 16), lambda i: (i, 0)),
            pl.BlockSpec((FB, D_IN), lambda i: (i, 0)),
            pl.BlockSpec((D_IN, D_OUT), lambda i: (0, 0)),
            pl.BlockSpec((1, D_OUT), lambda i: (0, 0)),
        ],
        out_specs=pl.BlockSpec((FB, D_OUT), lambda i: (i, 0)),
        out_shape=jax.ShapeDtypeStruct((N, D_OUT), f32),
    )(sums, cnts, nfeat, W_self, b_self2)
    return out


# speedup vs baseline: 3.4275x; 1.0214x over previous
import functools

import jax
import jax.numpy as jnp
from jax import lax
from jax.experimental import pallas as pl
from jax.experimental.pallas import tpu as pltpu
from jax.experimental.pallas import tpu_sc as plsc

N = 10000
E = 320000
D_IN = 128
D_EDGE = 16
D_OUT = 128
DH = D_OUT // 2
PACK = 8
QW = PACK * DH

NPAD = 10240
NBLK = NPAD // 128
CHUNK = 128
QROWS = CHUNK // PACK
NCHUNKS = E // CHUNK
NSUB = 16


def _p_body(nf_ref, wn_ref, p_ref):
    p_ref[...] = jnp.dot(nf_ref[...], wn_ref[:D_IN, :],
                         preferred_element_type=jnp.float32)


def _q_body(ea_ref, we_ref, wn_ref, bn_ref, q_ref):
    wf = jnp.dot(we_ref[...], wn_ref[0],
                 preferred_element_type=jnp.float32)
    z = jnp.zeros((D_EDGE, DH), jnp.float32)
    rows = []
    for j in range(PACK):
        pieces = [wf if t == j else z for t in range(PACK)]
        rows.append(jnp.concatenate(pieces, axis=1))
    wbig = jnp.concatenate(rows, axis=0)
    bias = jnp.concatenate([bn_ref[0]] * PACK, axis=1)
    q_ref[0] = jnp.dot(ea_ref[...], wbig,
                       preferred_element_type=jnp.float32) + bias


def _fin_body(sums_ref, cnts_ref, nf_ref, ws_ref, bs_ref, o_ref):
    s = sums_ref[...]
    tot = jnp.concatenate([s[0], s[1]], axis=1)
    c = cnts_ref[...]
    cnt = (c[0] + c[1])[:, 0:1]
    neigh = tot / jnp.maximum(cnt, 1.0)
    o_ref[...] = neigh + jnp.dot(nf_ref[...], ws_ref[...],
                                 preferred_element_type=jnp.float32) + bs_ref[...]


def _sc_edge_kernel(ei3, p_hbm, q_hbm, sums_out, cnts_out,
                    idx0, idx1, pb0, pb1, qb0, qb1, obuf, p_spm, acc_s, acc_c,
                    sg0, sg1, sq0, sq1, ss0, ss1, sc0, sc1):
    cid = lax.axis_index("c")
    sid = lax.axis_index("s")

    idxs = (idx0, idx1)
    pbs = (pb0, pb1)
    qbs = (qb0, qb1)
    sgs = (sg0, sg1)
    sqs = (sq0, sq1)
    sss = (ss0, ss1)
    scs = (sc0, sc1)

    nmy = (NCHUNKS - sid + NSUB - 1) // NSUB

    def _zrow(r, _):
        for cc in range(DH // 16):
            pb0[r, pl.ds(cc * 16, 16)] = jnp.zeros((16,), jnp.float32)
        obuf[r, pl.ds(0, 16)] = jnp.zeros((16,), jnp.float32)
        return 0
    lax.fori_loop(0, CHUNK, _zrow, 0)

    def _zblk(k, _):
        b = sid + NSUB * k
        pltpu.sync_copy(pb0, acc_s.at[pl.ds(b * 128, 128)])
        pltpu.sync_copy(obuf, acc_c.at[pl.ds(b * 128, 128)])
        pltpu.sync_copy(p_hbm.at[pl.ds(b * 128, 128), pl.ds(cid * DH, DH)],
                        p_spm.at[pl.ds(b * 128, 128)])
        return 0
    lax.fori_loop(0, NBLK // NSUB, _zblk, 0)
    plsc.subcore_barrier()

    def _orow(r, _):
        obuf[r, pl.ds(0, 16)] = jnp.ones((16,), jnp.float32)
        return 0
    lax.fori_loop(0, CHUNK, _orow, 0)

    def _load_idx(s, j):
        ch = sid + j * NSUB
        pltpu.sync_copy(ei3.at[0, ch], idxs[s].at[0])
        pltpu.sync_copy(ei3.at[1, ch], idxs[s].at[1])

    def _start_fetch(s, j):
        ch = sid + j * NSUB
        pltpu.async_copy(p_spm.at[idxs[s].at[0]], pbs[s], sgs[s])
        pltpu.async_copy(q_hbm.at[cid, pl.ds(ch * QROWS, QROWS)],
                         qbs[s], sqs[s])

    def _wait_fetch(s):
        pltpu.make_async_copy(p_spm.at[idxs[s].at[0]], pbs[s], sgs[s]).wait()
        pltpu.make_async_copy(q_hbm.at[cid, pl.ds(0, QROWS)], qbs[s], sqs[s]).wait()

    def _start_scatter(s, j):
        pltpu.async_copy(pbs[s], acc_s.at[idxs[s].at[1]], sss[s], add=True)

        @pl.when(cid == s)
        def _():
            pltpu.async_copy(obuf, acc_c.at[idxs[s].at[1]], scs[s], add=True)

    def _wait_scatter(s):
        pltpu.make_async_copy(pbs[s], acc_s.at[idxs[s].at[1]], sss[s]).wait()

        @pl.when(cid == s)
        def _():
            pltpu.make_async_copy(obuf, acc_c.at[idxs[s].at[1]], scs[s]).wait()

    def _compute(s):
        pb, qb = pbs[s], qbs[s]

        @plsc.parallel_loop(0, CHUNK, 1, unroll=8)
        def _crow(r):
            qrow = r >> 3
            qcol = (r & 7) * DH
            for cc in range(DH // 16):
                g = pl.ds(cc * 16, 16)
                pb[r, g] = jnp.maximum(
                    pb[r, g] + qb[qrow, pl.ds(qcol + cc * 16, 16)], 0.0)

    npairs = (nmy + 1) // 2

    _load_idx(0, 0)
    _start_fetch(0, 0)

    def _pair(k, _):
        for s in (0, 1):
            j = 2 * k + s
            nxt = 1 - s

            @pl.when(j + 1 < nmy)
            def _():
                @pl.when(j >= 1)
                def _():
                    _wait_scatter(nxt)
                _load_idx(nxt, j + 1)
                _start_fetch(nxt, j + 1)

            @pl.when(j < nmy)
            def _():
                _wait_fetch(s)
                _compute(s)
                _start_scatter(s, j)
        return 0
    lax.fori_loop(0, npairs, _pair, 0)

    _wait_scatter(0)
    _wait_scatter(1)
    plsc.subcore_barrier()

    def _dblk(k, _):
        b = sid + NSUB * k
        pltpu.sync_copy(acc_s.at[pl.ds(b * 128, 128)],
                        sums_out.at[cid, pl.ds(b * 128, 128)])

        pltpu.sync_copy(acc_c.at[pl.ds(b * 128, 128)],
                        cnts_out.at[cid, pl.ds(b * 128, 128)])
        return 0
    lax.fori_loop(0, NBLK // NSUB, _dblk, 0)


def kernel(nfeat, edge_index, edge_attr, W_edge, W_ne, b_ne, W_self, b_self):
    f32 = jnp.float32
    b_self2 = b_self.reshape(1, D_OUT)
    wn_bot_h = jnp.stack([W_ne[D_IN:, :DH], W_ne[D_IN:, DH:]])
    bn_h = jnp.stack([b_ne[:DH], b_ne[DH:]]).reshape(2, 1, DH)
    ea8 = edge_attr.reshape(E // PACK, PACK * D_EDGE)
    ei3 = edge_index.reshape(2, NCHUNKS, CHUNK)

    PB = 400
    P = pl.pallas_call(
        _p_body,
        grid=(N // PB,),
        in_specs=[
            pl.BlockSpec((PB, D_IN), lambda i: (i, 0)),
            pl.BlockSpec((D_IN + D_OUT, D_OUT), lambda i: (0, 0)),
        ],
        out_specs=pl.BlockSpec((PB, D_OUT), lambda i: (i, 0)),
        out_shape=jax.ShapeDtypeStruct((NPAD, D_OUT), f32),
    )(nfeat, W_ne)

    QB = 2000
    Q = pl.pallas_call(
        _q_body,
        grid=(2, E // PACK // QB),
        in_specs=[
            pl.BlockSpec((QB, PACK * D_EDGE), lambda h, i: (i, 0)),
            pl.BlockSpec((D_EDGE, D_OUT), lambda h, i: (0, 0)),
            pl.BlockSpec((1, D_OUT, DH), lambda h, i: (h, 0, 0)),
            pl.BlockSpec((1, 1, DH), lambda h, i: (h, 0, 0)),
        ],
        out_specs=pl.BlockSpec((1, QB, QW), lambda h, i: (h, i, 0)),
        out_shape=jax.ShapeDtypeStruct((2, E // PACK, QW), f32),
    )(ea8, W_edge, wn_bot_h, bn_h)

    mesh = plsc.VectorSubcoreMesh(core_axis_name="c", subcore_axis_name="s")
    sc = functools.partial(
        pl.kernel,
        mesh=mesh,
        compiler_params=pltpu.CompilerParams(use_tc_tiling_on_sc=False),
        out_type=(
            jax.ShapeDtypeStruct((2, NPAD, DH), f32),
            jax.ShapeDtypeStruct((2, NPAD, 16), f32),
        ),
        scratch_types=[
            pltpu.VMEM((2, CHUNK), jnp.int32),
            pltpu.VMEM((2, CHUNK), jnp.int32),
            pltpu.VMEM((CHUNK, DH), f32),
            pltpu.VMEM((CHUNK, DH), f32),
            pltpu.VMEM((QROWS, QW), f32),
            pltpu.VMEM((QROWS, QW), f32),
            pltpu.VMEM((CHUNK, 16), f32),
            pltpu.VMEM_SHARED((NPAD, DH), f32),
            pltpu.VMEM_SHARED((NPAD, DH), f32),
            pltpu.VMEM_SHARED((NPAD, 16), f32),
            pltpu.SemaphoreType.DMA,
            pltpu.SemaphoreType.DMA,
            pltpu.SemaphoreType.DMA,
            pltpu.SemaphoreType.DMA,
            pltpu.SemaphoreType.DMA,
            pltpu.SemaphoreType.DMA,
            pltpu.SemaphoreType.DMA,
            pltpu.SemaphoreType.DMA,
        ],
    )(_sc_edge_kernel)
    sums, cnts = sc(ei3, P, Q)

    FB = 400
    out = pl.pallas_call(
        _fin_body,
        grid=(N // FB,),
        in_specs=[
            pl.BlockSpec((2, FB, DH), lambda i: (0, i, 0)),
            pl.BlockSpec((2, FB, 16), lambda i: (0, i, 0)),
            pl.BlockSpec((FB, D_IN), lambda i: (i, 0)),
            pl.BlockSpec((D_IN, D_OUT), lambda i: (0, 0)),
            pl.BlockSpec((1, D_OUT), lambda i: (0, 0)),
        ],
        out_specs=pl.BlockSpec((FB, D_OUT), lambda i: (i, 0)),
        out_shape=jax.ShapeDtypeStruct((N, D_OUT), f32),
    )(sums, cnts, nfeat, W_self, b_self2)
    return out


# speedup vs baseline: 3.4444x; 1.0049x over previous
import functools

import jax
import jax.numpy as jnp
from jax import lax
from jax.experimental import pallas as pl
from jax.experimental.pallas import tpu as pltpu
from jax.experimental.pallas import tpu_sc as plsc

N = 10000
E = 320000
D_IN = 128
D_EDGE = 16
D_OUT = 128
DH = D_OUT // 2
PACK = 8
QW = PACK * DH

NPAD = 10240
NBLK = NPAD // 128
CHUNK = 128
QROWS = CHUNK // PACK
NCHUNKS = E // CHUNK
NSUB = 16


def _p_body(nf_ref, wn_ref, p_ref):
    p_ref[...] = jnp.dot(nf_ref[...], wn_ref[0],
                         preferred_element_type=jnp.float32)


def _q_body(ea_ref, we_ref, wn_ref, bn_ref, q_ref):
    wf = jnp.dot(we_ref[...], wn_ref[0],
                 preferred_element_type=jnp.float32)
    z = jnp.zeros((D_EDGE, DH), jnp.float32)
    rows = []
    for j in range(PACK):
        pieces = [wf if t == j else z for t in range(PACK)]
        rows.append(jnp.concatenate(pieces, axis=1))
    wbig = jnp.concatenate(rows, axis=0)
    bias = jnp.concatenate([bn_ref[0]] * PACK, axis=1)
    q_ref[0] = jnp.dot(ea_ref[...], wbig,
                       preferred_element_type=jnp.float32) + bias


def _fin_body(sums_ref, cnts_ref, nf_ref, ws_ref, bs_ref, o_ref):
    s = sums_ref[...]
    tot = jnp.concatenate([s[0], s[1]], axis=1)
    c = cnts_ref[...]
    cnt = (c[0] + c[1])[:, 0:1]
    neigh = tot / jnp.maximum(cnt, 1.0)
    o_ref[...] = neigh + jnp.dot(nf_ref[...], ws_ref[...],
                                 preferred_element_type=jnp.float32) + bs_ref[...]


def _sc_edge_kernel(ei3, p_hbm, q_hbm, sums_out, cnts_out,
                    idx0, idx1, pb0, pb1, qb0, qb1, obuf, acc_s, acc_c,
                    sg0, sg1, sq0, sq1, ss0, ss1, sc0, sc1):
    cid = lax.axis_index("c")
    sid = lax.axis_index("s")

    idxs = (idx0, idx1)
    pbs = (pb0, pb1)
    qbs = (qb0, qb1)
    sgs = (sg0, sg1)
    sqs = (sq0, sq1)
    sss = (ss0, ss1)
    scs = (sc0, sc1)

    nmy = (NCHUNKS - sid + NSUB - 1) // NSUB

    def _zrow(r, _):
        for cc in range(DH // 16):
            pb0[r, pl.ds(cc * 16, 16)] = jnp.zeros((16,), jnp.float32)
        obuf[r, pl.ds(0, 16)] = jnp.zeros((16,), jnp.float32)
        return 0
    lax.fori_loop(0, CHUNK, _zrow, 0)

    def _zblk(k, _):
        b = sid + NSUB * k
        pltpu.sync_copy(pb0, acc_s.at[pl.ds(b * 128, 128)])
        pltpu.sync_copy(obuf, acc_c.at[pl.ds(b * 128, 128)])
        return 0
    lax.fori_loop(0, NBLK // NSUB, _zblk, 0)
    plsc.subcore_barrier()

    def _orow(r, _):
        obuf[r, pl.ds(0, 16)] = jnp.ones((16,), jnp.float32)
        return 0
    lax.fori_loop(0, CHUNK, _orow, 0)

    off = cid * N

    def _load_idx(s, j):
        ch = sid + j * NSUB
        pltpu.sync_copy(ei3.at[0, ch], idxs[s].at[0])
        pltpu.sync_copy(ei3.at[1, ch], idxs[s].at[1])
        for g in range(CHUNK // 16):
            gsl = pl.ds(g * 16, 16)
            idxs[s][0, gsl] = idxs[s][0, gsl] + off

    def _start_fetch(s, j):
        ch = sid + j * NSUB
        pltpu.async_copy(p_hbm.at[idxs[s].at[0]], pbs[s], sgs[s])
        pltpu.async_copy(q_hbm.at[cid, pl.ds(ch * QROWS, QROWS)],
                         qbs[s], sqs[s])

    def _wait_fetch(s):
        pltpu.make_async_copy(p_hbm.at[idxs[s].at[0]], pbs[s], sgs[s]).wait()
        pltpu.make_async_copy(q_hbm.at[cid, pl.ds(0, QROWS)], qbs[s], sqs[s]).wait()

    def _start_scatter(s, j):
        pltpu.async_copy(pbs[s], acc_s.at[idxs[s].at[1]], sss[s], add=True)

        @pl.when(cid == s)
        def _():
            pltpu.async_copy(obuf, acc_c.at[idxs[s].at[1]], scs[s], add=True)

    def _wait_scatter(s):
        pltpu.make_async_copy(pbs[s], acc_s.at[idxs[s].at[1]], sss[s]).wait()

        @pl.when(cid == s)
        def _():
            pltpu.make_async_copy(obuf, acc_c.at[idxs[s].at[1]], scs[s]).wait()

    def _compute(s):
        pb, qb = pbs[s], qbs[s]

        @plsc.parallel_loop(0, CHUNK, 1, unroll=8)
        def _crow(r):
            qrow = r >> 3
            qcol = (r & 7) * DH
            for cc in range(DH // 16):
                g = pl.ds(cc * 16, 16)
                pb[r, g] = jnp.maximum(
                    pb[r, g] + qb[qrow, pl.ds(qcol + cc * 16, 16)], 0.0)

    npairs = (nmy + 1) // 2

    _load_idx(0, 0)
    _start_fetch(0, 0)

    def _pair(k, _):
        for s in (0, 1):
            j = 2 * k + s
            nxt = 1 - s

            @pl.when(j + 1 < nmy)
            def _():
                @pl.when(j >= 1)
                def _():
                    _wait_scatter(nxt)
                _load_idx(nxt, j + 1)
                _start_fetch(nxt, j + 1)

            @pl.when(j < nmy)
            def _():
                _wait_fetch(s)
                _compute(s)
                _start_scatter(s, j)
        return 0
    lax.fori_loop(0, npairs, _pair, 0)

    _wait_scatter(0)
    _wait_scatter(1)
    plsc.subcore_barrier()

    def _dblk(k, _):
        b = sid + NSUB * k
        pltpu.sync_copy(acc_s.at[pl.ds(b * 128, 128)],
                        sums_out.at[cid, pl.ds(b * 128, 128)])

        pltpu.sync_copy(acc_c.at[pl.ds(b * 128, 128)],
                        cnts_out.at[cid, pl.ds(b * 128, 128)])
        return 0
    lax.fori_loop(0, NBLK // NSUB, _dblk, 0)


def kernel(nfeat, edge_index, edge_attr, W_edge, W_ne, b_ne, W_self, b_self):
    f32 = jnp.float32
    b_self2 = b_self.reshape(1, D_OUT)
    wn_top_h = jnp.stack([W_ne[:D_IN, :DH], W_ne[:D_IN, DH:]])
    wn_bot_h = jnp.stack([W_ne[D_IN:, :DH], W_ne[D_IN:, DH:]])
    bn_h = jnp.stack([b_ne[:DH], b_ne[DH:]]).reshape(2, 1, DH)
    ea8 = edge_attr.reshape(E // PACK, PACK * D_EDGE)
    ei3 = edge_index.reshape(2, NCHUNKS, CHUNK)

    PB = 400
    P = pl.pallas_call(
        _p_body,
        grid=(2, N // PB),
        in_specs=[
            pl.BlockSpec((PB, D_IN), lambda h, i: (i, 0)),
            pl.BlockSpec((1, D_IN, DH), lambda h, i: (h, 0, 0)),
        ],
        out_specs=pl.BlockSpec((PB, DH), lambda h, i: (h * (N // PB) + i, 0)),
        out_shape=jax.ShapeDtypeStruct((2 * N, DH), f32),
    )(nfeat, wn_top_h)

    QB = 2000
    Q = pl.pallas_call(
        _q_body,
        grid=(2, E // PACK // QB),
        in_specs=[
            pl.BlockSpec((QB, PACK * D_EDGE), lambda h, i: (i, 0)),
            pl.BlockSpec((D_EDGE, D_OUT), lambda h, i: (0, 0)),
            pl.BlockSpec((1, D_OUT, DH), lambda h, i: (h, 0, 0)),
            pl.BlockSpec((1, 1, DH), lambda h, i: (h, 0, 0)),
        ],
        out_specs=pl.BlockSpec((1, QB, QW), lambda h, i: (h, i, 0)),
        out_shape=jax.ShapeDtypeStruct((2, E // PACK, QW), f32),
    )(ea8, W_edge, wn_bot_h, bn_h)

    mesh = plsc.VectorSubcoreMesh(core_axis_name="c", subcore_axis_name="s")
    sc = functools.partial(
        pl.kernel,
        mesh=mesh,
        compiler_params=pltpu.CompilerParams(use_tc_tiling_on_sc=False),
        out_type=(
            jax.ShapeDtypeStruct((2, NPAD, DH), f32),
            jax.ShapeDtypeStruct((2, NPAD, 16), f32),
        ),
        scratch_types=[
            pltpu.VMEM((2, CHUNK), jnp.int32),
            pltpu.VMEM((2, CHUNK), jnp.int32),
            pltpu.VMEM((CHUNK, DH), f32),
            pltpu.VMEM((CHUNK, DH), f32),
            pltpu.VMEM((QROWS, QW), f32),
            pltpu.VMEM((QROWS, QW), f32),
            pltpu.VMEM((CHUNK, 16), f32),
            pltpu.VMEM_SHARED((NPAD, DH), f32),
            pltpu.VMEM_SHARED((NPAD, 16), f32),
            pltpu.SemaphoreType.DMA,
            pltpu.SemaphoreType.DMA,
            pltpu.SemaphoreType.DMA,
            pltpu.SemaphoreType.DMA,
            pltpu.SemaphoreType.DMA,
            pltpu.SemaphoreType.DMA,
            pltpu.SemaphoreType.DMA,
            pltpu.SemaphoreType.DMA,
        ],
    )(_sc_edge_kernel)
    sums, cnts = sc(ei3, P, Q)

    FB = 400
    out = pl.pallas_call(
        _fin_body,
        grid=(N // FB,),
        in_specs=[
            pl.BlockSpec((2, FB, DH), lambda i: (0, i, 0)),
            pl.BlockSpec((2, FB, 16), lambda i: (0, i, 0)),
            pl.BlockSpec((FB, D_IN), lambda i: (i, 0)),
            pl.BlockSpec((D_IN, D_OUT), lambda i: (0, 0)),
            pl.BlockSpec((1, D_OUT), lambda i: (0, 0)),
        ],
        out_specs=pl.BlockSpec((FB, D_OUT), lambda i: (i, 0)),
        out_shape=jax.ShapeDtypeStruct((N, D_OUT), f32),
    )(sums, cnts, nfeat, W_self, b_self2)
    return out


# speedup vs baseline: 3.8195x; 1.1089x over previous
import functools

import jax
import jax.numpy as jnp
from jax import lax
from jax.experimental import pallas as pl
from jax.experimental.pallas import tpu as pltpu
from jax.experimental.pallas import tpu_sc as plsc

N = 10000
E = 320000
D_IN = 128
D_EDGE = 16
D_OUT = 128
DH = D_OUT // 2
PACK = 8
QW = PACK * DH

NPAD = 10240
NBLK = NPAD // 128
CHUNK = 256
QROWS = CHUNK // PACK
NCHUNKS = E // CHUNK
IROW = 128
NSUB = 16


def _p_body(nf_ref, wn_ref, p_ref):
    p_ref[...] = jnp.dot(nf_ref[...], wn_ref[0],
                         preferred_element_type=jnp.float32)


def _q_body(ea_ref, we_ref, wn_ref, bn_ref, q_ref):
    wf = jnp.dot(we_ref[...], wn_ref[0],
                 preferred_element_type=jnp.float32)
    z = jnp.zeros((D_EDGE, DH), jnp.float32)
    rows = []
    for j in range(PACK):
        pieces = [wf if t == j else z for t in range(PACK)]
        rows.append(jnp.concatenate(pieces, axis=1))
    wbig = jnp.concatenate(rows, axis=0)
    bias = jnp.concatenate([bn_ref[0]] * PACK, axis=1)
    q_ref[0] = jnp.dot(ea_ref[...], wbig,
                       preferred_element_type=jnp.float32) + bias


def _fin_body(sums_ref, cnts_ref, nf_ref, ws_ref, bs_ref, o_ref):
    s = sums_ref[...]
    tot = jnp.concatenate([s[0], s[1]], axis=1)
    c = cnts_ref[...]
    cnt = (c[0] + c[1])[:, 0:1]
    neigh = tot / jnp.maximum(cnt, 1.0)
    o_ref[...] = neigh + jnp.dot(nf_ref[...], ws_ref[...],
                                 preferred_element_type=jnp.float32) + bs_ref[...]


def _sc_edge_kernel(ei3, p_hbm, q_hbm, sums_out, cnts_out,
                    idx0, idx1, pb0, pb1, qb0, qb1, obuf, acc_s, acc_c,
                    sg0, sg1, sq0, sq1, ss0, ss1, sc0, sc1):
    cid = lax.axis_index("c")
    sid = lax.axis_index("s")

    idxs = (idx0, idx1)
    pbs = (pb0, pb1)
    qbs = (qb0, qb1)
    sgs = (sg0, sg1)
    sqs = (sq0, sq1)
    sss = (ss0, ss1)
    scs = (sc0, sc1)

    nmy = (NCHUNKS - sid + NSUB - 1) // NSUB

    def _zrow(r, _):
        for cc in range(DH // 16):
            pb0[r, pl.ds(cc * 16, 16)] = jnp.zeros((16,), jnp.float32)
        return 0
    lax.fori_loop(0, CHUNK, _zrow, 0)

    def _zorow(r, _):
        obuf[r, pl.ds(0, 16)] = jnp.zeros((16,), jnp.float32)
        return 0
    lax.fori_loop(0, IROW, _zorow, 0)

    def _zblk(k, _):
        b = sid + NSUB * k
        pltpu.sync_copy(pb0.at[pl.ds(0, 128)], acc_s.at[pl.ds(b * 128, 128)])
        pltpu.sync_copy(obuf, acc_c.at[pl.ds(b * 128, 128)])
        return 0
    lax.fori_loop(0, NBLK // NSUB, _zblk, 0)
    plsc.subcore_barrier()

    def _orow(r, _):
        obuf[r, pl.ds(0, 16)] = jnp.ones((16,), jnp.float32)
        return 0
    lax.fori_loop(0, IROW, _orow, 0)

    off = cid * N

    def _load_idx(s, j):
        ch = sid + j * NSUB
        pltpu.sync_copy(ei3.at[0, pl.ds(2 * ch, 2)], idxs[s].at[pl.ds(0, 2)])
        pltpu.sync_copy(ei3.at[1, pl.ds(2 * ch, 2)], idxs[s].at[pl.ds(2, 2)])
        for h in range(2):
            for g in range(IROW // 16):
                gsl = pl.ds(g * 16, 16)
                idxs[s][h, gsl] = idxs[s][h, gsl] + off

    def _start_fetch(s, j):
        ch = sid + j * NSUB
        for h in range(2):
            pltpu.async_copy(p_hbm.at[idxs[s].at[h]],
                             pbs[s].at[pl.ds(h * IROW, IROW)], sgs[s])
        pltpu.async_copy(q_hbm.at[cid, pl.ds(ch * QROWS, QROWS)],
                         qbs[s], sqs[s])

    def _wait_fetch(s):
        for h in range(2):
            pltpu.make_async_copy(p_hbm.at[idxs[s].at[h]],
                                  pbs[s].at[pl.ds(h * IROW, IROW)], sgs[s]).wait()
        pltpu.make_async_copy(q_hbm.at[cid, pl.ds(0, QROWS)], qbs[s], sqs[s]).wait()

    def _start_scatter(s, j):
        for h in range(2):
            pltpu.async_copy(pbs[s].at[pl.ds(h * IROW, IROW)],
                             acc_s.at[idxs[s].at[2 + h]], sss[s], add=True)

        @pl.when(cid == s)
        def _():
            for h in range(2):
                pltpu.async_copy(obuf, acc_c.at[idxs[s].at[2 + h]],
                                 scs[s], add=True)

    def _wait_scatter(s):
        for h in range(2):
            pltpu.make_async_copy(pbs[s].at[pl.ds(h * IROW, IROW)],
                                  acc_s.at[idxs[s].at[2 + h]], sss[s]).wait()

        @pl.when(cid == s)
        def _():
            for h in range(2):
                pltpu.make_async_copy(obuf, acc_c.at[idxs[s].at[2 + h]],
                                      scs[s]).wait()

    def _compute(s):
        pb, qb = pbs[s], qbs[s]

        @plsc.parallel_loop(0, CHUNK, 1, unroll=8)
        def _crow(r):
            qrow = r >> 3
            qcol = (r & 7) * DH
            for cc in range(DH // 16):
                g = pl.ds(cc * 16, 16)
                pb[r, g] = jnp.maximum(
                    pb[r, g] + qb[qrow, pl.ds(qcol + cc * 16, 16)], 0.0)

    npairs = (nmy + 1) // 2

    _load_idx(0, 0)
    _start_fetch(0, 0)

    def _pair(k, _):
        for s in (0, 1):
            j = 2 * k + s
            nxt = 1 - s

            @pl.when(j + 1 < nmy)
            def _():
                @pl.when(j >= 1)
                def _():
                    _wait_scatter(nxt)
                _load_idx(nxt, j + 1)
                _start_fetch(nxt, j + 1)

            @pl.when(j < nmy)
            def _():
                _wait_fetch(s)
                _compute(s)
                _start_scatter(s, j)
        return 0
    lax.fori_loop(0, npairs, _pair, 0)

    _wait_scatter(0)
    _wait_scatter(1)
    plsc.subcore_barrier()

    def _dblk(k, _):
        b = sid + NSUB * k
        pltpu.sync_copy(acc_s.at[pl.ds(b * 128, 128)],
                        sums_out.at[cid, pl.ds(b * 128, 128)])

        pltpu.sync_copy(acc_c.at[pl.ds(b * 128, 128)],
                        cnts_out.at[cid, pl.ds(b * 128, 128)])
        return 0
    lax.fori_loop(0, NBLK // NSUB, _dblk, 0)


def kernel(nfeat, edge_index, edge_attr, W_edge, W_ne, b_ne, W_self, b_self):
    f32 = jnp.float32
    b_self2 = b_self.reshape(1, D_OUT)
    wn_top_h = jnp.stack([W_ne[:D_IN, :DH], W_ne[:D_IN, DH:]])
    wn_bot_h = jnp.stack([W_ne[D_IN:, :DH], W_ne[D_IN:, DH:]])
    bn_h = jnp.stack([b_ne[:DH], b_ne[DH:]]).reshape(2, 1, DH)
    ea8 = edge_attr.reshape(E // PACK, PACK * D_EDGE)
    ei3 = edge_index.reshape(2, E // IROW, IROW)

    PB = 400
    P = pl.pallas_call(
        _p_body,
        grid=(2, N // PB),
        in_specs=[
            pl.BlockSpec((PB, D_IN), lambda h, i: (i, 0)),
            pl.BlockSpec((1, D_IN, DH), lambda h, i: (h, 0, 0)),
        ],
        out_specs=pl.BlockSpec((PB, DH), lambda h, i: (h * (N // PB) + i, 0)),
        out_shape=jax.ShapeDtypeStruct((2 * N, DH), f32),
    )(nfeat, wn_top_h)

    QB = 2000
    Q = pl.pallas_call(
        _q_body,
        grid=(2, E // PACK // QB),
        in_specs=[
            pl.BlockSpec((QB, PACK * D_EDGE), lambda h, i: (i, 0)),
            pl.BlockSpec((D_EDGE, D_OUT), lambda h, i: (0, 0)),
            pl.BlockSpec((1, D_OUT, DH), lambda h, i: (h, 0, 0)),
            pl.BlockSpec((1, 1, DH), lambda h, i: (h, 0, 0)),
        ],
        out_specs=pl.BlockSpec((1, QB, QW), lambda h, i: (h, i, 0)),
        out_shape=jax.ShapeDtypeStruct((2, E // PACK, QW), f32),
    )(ea8, W_edge, wn_bot_h, bn_h)

    mesh = plsc.VectorSubcoreMesh(core_axis_name="c", subcore_axis_name="s")
    sc = functools.partial(
        pl.kernel,
        mesh=mesh,
        compiler_params=pltpu.CompilerParams(use_tc_tiling_on_sc=False),
        out_type=(
            jax.ShapeDtypeStruct((2, NPAD, DH), f32),
            jax.ShapeDtypeStruct((2, NPAD, 16), f32),
        ),
        scratch_types=[
            pltpu.VMEM((4, IROW), jnp.int32),
            pltpu.VMEM((4, IROW), jnp.int32),
            pltpu.VMEM((CHUNK, DH), f32),
            pltpu.VMEM((CHUNK, DH), f32),
            pltpu.VMEM((QROWS, QW), f32),
            pltpu.VMEM((QROWS, QW), f32),
            pltpu.VMEM((IROW, 16), f32),
            pltpu.VMEM_SHARED((NPAD, DH), f32),
            pltpu.VMEM_SHARED((NPAD, 16), f32),
            pltpu.SemaphoreType.DMA,
            pltpu.SemaphoreType.DMA,
            pltpu.SemaphoreType.DMA,
            pltpu.SemaphoreType.DMA,
            pltpu.SemaphoreType.DMA,
            pltpu.SemaphoreType.DMA,
            pltpu.SemaphoreType.DMA,
            pltpu.SemaphoreType.DMA,
        ],
    )(_sc_edge_kernel)
    sums, cnts = sc(ei3, P, Q)

    FB = 400
    out = pl.pallas_call(
        _fin_body,
        grid=(N // FB,),
        in_specs=[
            pl.BlockSpec((2, FB, DH), lambda i: (0, i, 0)),
            pl.BlockSpec((2, FB, 16), lambda i: (0, i, 0)),
            pl.BlockSpec((FB, D_IN), lambda i: (i, 0)),
            pl.BlockSpec((D_IN, D_OUT), lambda i: (0, 0)),
            pl.BlockSpec((1, D_OUT), lambda i: (0, 0)),
        ],
        out_specs=pl.BlockSpec((FB, D_OUT), lambda i: (i, 0)),
        out_shape=jax.ShapeDtypeStruct((N, D_OUT), f32),
    )(sums, cnts, nfeat, W_self, b_self2)
    return out


# speedup vs baseline: 3.8602x; 1.0106x over previous
import functools

import jax
import jax.numpy as jnp
from jax import lax
from jax.experimental import pallas as pl
from jax.experimental.pallas import tpu as pltpu
from jax.experimental.pallas import tpu_sc as plsc

N = 10000
E = 320000
D_IN = 128
D_EDGE = 16
D_OUT = 128
DH = D_OUT // 2
PACK = 8
QW = PACK * DH

NPAD = 10240
NBLK = NPAD // 128
CHUNK = 256
QROWS = CHUNK // PACK
NCHUNKS = E // CHUNK
IROW = 128
NSUB = 16


def _p_body(nf_ref, wn_ref, p_ref):
    p_ref[...] = jnp.dot(nf_ref[...], wn_ref[:D_IN, :],
                         preferred_element_type=jnp.float32)


def _q_body(ea_ref, we_ref, wn_ref, bn_ref, q_ref):
    wf = jnp.dot(we_ref[...], wn_ref[0],
                 preferred_element_type=jnp.float32)
    z = jnp.zeros((D_EDGE, DH), jnp.float32)
    rows = []
    for j in range(PACK):
        pieces = [wf if t == j else z for t in range(PACK)]
        rows.append(jnp.concatenate(pieces, axis=1))
    wbig = jnp.concatenate(rows, axis=0)
    bias = jnp.concatenate([bn_ref[0]] * PACK, axis=1)
    q_ref[0] = jnp.dot(ea_ref[...], wbig,
                       preferred_element_type=jnp.float32) + bias


def _fin_body(sums_ref, cnts_ref, nf_ref, ws_ref, bs_ref, o_ref):
    s = sums_ref[...]
    tot = jnp.concatenate([s[0], s[1]], axis=1)
    c = cnts_ref[...]
    cnt = (c[0] + c[1])[:, 0:1]
    neigh = tot / jnp.maximum(cnt, 1.0)
    o_ref[...] = neigh + jnp.dot(nf_ref[...], ws_ref[...],
                                 preferred_element_type=jnp.float32) + bs_ref[...]


def _sc_edge_kernel(ei3, p_hbm, q_hbm, sums_out, cnts_out,
                    idx0, idx1, pb0, pb1, qb0, qb1, obuf, acc_s, acc_c,
                    sg0, sg1, sq0, sq1, ss0, ss1, sc0, sc1):
    cid = lax.axis_index("c")
    sid = lax.axis_index("s")

    idxs = (idx0, idx1)
    pbs = (pb0, pb1)
    qbs = (qb0, qb1)
    sgs = (sg0, sg1)
    sqs = (sq0, sq1)
    sss = (ss0, ss1)
    scs = (sc0, sc1)

    nmy = (NCHUNKS - sid + NSUB - 1) // NSUB

    def _zrow(r, _):
        for cc in range(DH // 16):
            pb0[r, pl.ds(cc * 16, 16)] = jnp.zeros((16,), jnp.float32)
        return 0
    lax.fori_loop(0, CHUNK, _zrow, 0)

    def _zorow(r, _):
        obuf[r, pl.ds(0, 16)] = jnp.zeros((16,), jnp.float32)
        return 0
    lax.fori_loop(0, IROW, _zorow, 0)

    def _zblk(k, _):
        b = sid + NSUB * k
        pltpu.sync_copy(pb0.at[pl.ds(0, 128)], acc_s.at[pl.ds(b * 128, 128)])
        pltpu.sync_copy(obuf, acc_c.at[pl.ds(b * 128, 128)])
        return 0
    lax.fori_loop(0, NBLK // NSUB, _zblk, 0)
    plsc.subcore_barrier()

    def _orow(r, _):
        obuf[r, pl.ds(0, 16)] = jnp.ones((16,), jnp.float32)
        return 0
    lax.fori_loop(0, IROW, _orow, 0)

    def _load_idx(s, j):
        ch = sid + j * NSUB
        pltpu.sync_copy(ei3.at[0, pl.ds(2 * ch, 2)], idxs[s].at[pl.ds(0, 2)])
        pltpu.sync_copy(ei3.at[1, pl.ds(2 * ch, 2)], idxs[s].at[pl.ds(2, 2)])
        for h in range(2):
            for g in range(IROW // 16):
                gsl = pl.ds(g * 16, 16)
                idxs[s][h, gsl] = 2 * idxs[s][h, gsl] + cid

    def _start_fetch(s, j):
        ch = sid + j * NSUB
        for h in range(2):
            pltpu.async_copy(p_hbm.at[idxs[s].at[h]],
                             pbs[s].at[pl.ds(h * IROW, IROW)], sgs[s])
        pltpu.async_copy(q_hbm.at[cid, pl.ds(ch * QROWS, QROWS)],
                         qbs[s], sqs[s])

    def _wait_fetch(s):
        for h in range(2):
            pltpu.make_async_copy(p_hbm.at[idxs[s].at[h]],
                                  pbs[s].at[pl.ds(h * IROW, IROW)], sgs[s]).wait()
        pltpu.make_async_copy(q_hbm.at[cid, pl.ds(0, QROWS)], qbs[s], sqs[s]).wait()

    def _start_scatter(s, j):
        for h in range(2):
            pltpu.async_copy(pbs[s].at[pl.ds(h * IROW, IROW)],
                             acc_s.at[idxs[s].at[2 + h]], sss[s], add=True)

        @pl.when(cid == s)
        def _():
            for h in range(2):
                pltpu.async_copy(obuf, acc_c.at[idxs[s].at[2 + h]],
                                 scs[s], add=True)

    def _wait_scatter(s):
        for h in range(2):
            pltpu.make_async_copy(pbs[s].at[pl.ds(h * IROW, IROW)],
                                  acc_s.at[idxs[s].at[2 + h]], sss[s]).wait()

        @pl.when(cid == s)
        def _():
            for h in range(2):
                pltpu.make_async_copy(obuf, acc_c.at[idxs[s].at[2 + h]],
                                      scs[s]).wait()

    def _compute(s):
        pb, qb = pbs[s], qbs[s]

        @plsc.parallel_loop(0, CHUNK, 1, unroll=8)
        def _crow(r):
            qrow = r >> 3
            qcol = (r & 7) * DH
            for cc in range(DH // 16):
                g = pl.ds(cc * 16, 16)
                pb[r, g] = jnp.maximum(
                    pb[r, g] + qb[qrow, pl.ds(qcol + cc * 16, 16)], 0.0)

    npairs = (nmy + 1) // 2

    _load_idx(0, 0)
    _start_fetch(0, 0)

    def _pair(k, _):
        for s in (0, 1):
            j = 2 * k + s
            nxt = 1 - s

            @pl.when(j + 1 < nmy)
            def _():
                @pl.when(j >= 1)
                def _():
                    _wait_scatter(nxt)
                _load_idx(nxt, j + 1)
                _start_fetch(nxt, j + 1)

            @pl.when(j < nmy)
            def _():
                _wait_fetch(s)
                _compute(s)
                _start_scatter(s, j)
        return 0
    lax.fori_loop(0, npairs, _pair, 0)

    _wait_scatter(0)
    _wait_scatter(1)
    plsc.subcore_barrier()

    def _dblk(k, _):
        b = sid + NSUB * k
        pltpu.sync_copy(acc_s.at[pl.ds(b * 128, 128)],
                        sums_out.at[cid, pl.ds(b * 128, 128)])

        pltpu.sync_copy(acc_c.at[pl.ds(b * 128, 128)],
                        cnts_out.at[cid, pl.ds(b * 128, 128)])
        return 0
    lax.fori_loop(0, NBLK // NSUB, _dblk, 0)


def kernel(nfeat, edge_index, edge_attr, W_edge, W_ne, b_ne, W_self, b_self):
    f32 = jnp.float32
    b_self2 = b_self.reshape(1, D_OUT)
    wn_bot_h = jnp.stack([W_ne[D_IN:, :DH], W_ne[D_IN:, DH:]])
    bn_h = jnp.stack([b_ne[:DH], b_ne[DH:]]).reshape(2, 1, DH)
    ea8 = edge_attr.reshape(E // PACK, PACK * D_EDGE)
    ei3 = edge_index.reshape(2, E // IROW, IROW)

    PB = 400
    P = pl.pallas_call(
        _p_body,
        grid=(N // PB,),
        in_specs=[
            pl.BlockSpec((PB, D_IN), lambda i: (i, 0)),
            pl.BlockSpec((D_IN + D_OUT, D_OUT), lambda i: (0, 0)),
        ],
        out_specs=pl.BlockSpec((PB, D_OUT), lambda i: (i, 0)),
        out_shape=jax.ShapeDtypeStruct((N, D_OUT), f32),
    )(nfeat, W_ne).reshape(2 * N, DH)

    QB = 2000
    Q = pl.pallas_call(
        _q_body,
        grid=(2, E // PACK // QB),
        in_specs=[
            pl.BlockSpec((QB, PACK * D_EDGE), lambda h, i: (i, 0)),
            pl.BlockSpec((D_EDGE, D_OUT), lambda h, i: (0, 0)),
            pl.BlockSpec((1, D_OUT, DH), lambda h, i: (h, 0, 0)),
            pl.BlockSpec((1, 1, DH), lambda h, i: (h, 0, 0)),
        ],
        out_specs=pl.BlockSpec((1, QB, QW), lambda h, i: (h, i, 0)),
        out_shape=jax.ShapeDtypeStruct((2, E // PACK, QW), f32),
    )(ea8, W_edge, wn_bot_h, bn_h)

    mesh = plsc.VectorSubcoreMesh(core_axis_name="c", subcore_axis_name="s")
    sc = functools.partial(
        pl.kernel,
        mesh=mesh,
        compiler_params=pltpu.CompilerParams(use_tc_tiling_on_sc=False),
        out_type=(
            jax.ShapeDtypeStruct((2, NPAD, DH), f32),
            jax.ShapeDtypeStruct((2, NPAD, 16), f32),
        ),
        scratch_types=[
            pltpu.VMEM((4, IROW), jnp.int32),
            pltpu.VMEM((4, IROW), jnp.int32),
            pltpu.VMEM((CHUNK, DH), f32),
            pltpu.VMEM((CHUNK, DH), f32),
            pltpu.VMEM((QROWS, QW), f32),
            pltpu.VMEM((QROWS, QW), f32),
            pltpu.VMEM((IROW, 16), f32),
            pltpu.VMEM_SHARED((NPAD, DH), f32),
            pltpu.VMEM_SHARED((NPAD, 16), f32),
            pltpu.SemaphoreType.DMA,
            pltpu.SemaphoreType.DMA,
            pltpu.SemaphoreType.DMA,
            pltpu.SemaphoreType.DMA,
            pltpu.SemaphoreType.DMA,
            pltpu.SemaphoreType.DMA,
            pltpu.SemaphoreType.DMA,
            pltpu.SemaphoreType.DMA,
        ],
    )(_sc_edge_kernel)
    sums, cnts = sc(ei3, P, Q)

    FB = 400
    out = pl.pallas_call(
        _fin_body,
        grid=(N // FB,),
        in_specs=[
            pl.BlockSpec((2, FB, DH), lambda i: (0, i, 0)),
            pl.BlockSpec((2, FB, 16), lambda i: (0, i, 0)),
            pl.BlockSpec((FB, D_IN), lambda i: (i, 0)),
            pl.BlockSpec((D_IN, D_OUT), lambda i: (0, 0)),
            pl.BlockSpec((1, D_OUT), lambda i: (0, 0)),
        ],
        out_specs=pl.BlockSpec((FB, D_OUT), lambda i: (i, 0)),
        out_shape=jax.ShapeDtypeStruct((N, D_OUT), f32),
    )(sums, cnts, nfeat, W_self, b_self2)
    return out


# speedup vs baseline: 3.8617x; 1.0004x over previous
import functools

import jax
import jax.numpy as jnp
from jax import lax
from jax.experimental import pallas as pl
from jax.experimental.pallas import tpu as pltpu
from jax.experimental.pallas import tpu_sc as plsc

N = 10000
E = 320000
D_IN = 128
D_EDGE = 16
D_OUT = 128
DH = D_OUT // 2
PACK = 8
QW = PACK * DH

NPAD = 10240
NBLK = NPAD // 128
CHUNK = 256
QROWS = CHUNK // PACK
NCHUNKS = E // CHUNK
IROW = 128
NSUB = 16


def _p_body(nf_ref, wn_ref, p_ref):
    p_ref[...] = jnp.dot(nf_ref[...], wn_ref[:D_IN, :],
                         preferred_element_type=jnp.float32)


def _q_body(ea_ref, we_ref, wn_ref, bn_ref, q_ref):
    wf = jnp.dot(we_ref[...], wn_ref[0],
                 preferred_element_type=jnp.float32)
    z = jnp.zeros((D_EDGE, DH), jnp.float32)
    rows = []
    for j in range(PACK):
        pieces = [wf if t == j else z for t in range(PACK)]
        rows.append(jnp.concatenate(pieces, axis=1))
    wbig = jnp.concatenate(rows, axis=0)
    bias = jnp.concatenate([bn_ref[0]] * PACK, axis=1)
    q_ref[0] = jnp.dot(ea_ref[...], wbig,
                       preferred_element_type=jnp.float32) + bias


def _fin_body(sums_ref, cnts_ref, nf_ref, ws_ref, bs_ref, o_ref):
    s = sums_ref[...]
    tot = jnp.concatenate([s[0], s[1]], axis=1)
    c = cnts_ref[...]
    cnt = (c[0] + c[1])[:, 0:1]
    neigh = tot / jnp.maximum(cnt, 1.0)
    o_ref[...] = neigh + jnp.dot(nf_ref[...], ws_ref[...],
                                 preferred_element_type=jnp.float32) + bs_ref[...]


def _sc_edge_kernel(ei3, p_hbm, q_hbm, sums_out, cnts_out,
                    idx0, idx1, pb0, pb1, qb0, qb1, obuf, acc_s, acc_c,
                    sg0, sg1, sq0, sq1, ss0, ss1, sc0, sc1):
    cid = lax.axis_index("c")
    sid = lax.axis_index("s")

    idxs = (idx0, idx1)
    pbs = (pb0, pb1)
    qbs = (qb0, qb1)
    sgs = (sg0, sg1)
    sqs = (sq0, sq1)
    sss = (ss0, ss1)
    scs = (sc0, sc1)

    nmy = (NCHUNKS - sid + NSUB - 1) // NSUB

    def _zrow(r, _):
        for cc in range(DH // 16):
            pb0[r, pl.ds(cc * 16, 16)] = jnp.zeros((16,), jnp.float32)
        return 0
    lax.fori_loop(0, CHUNK, _zrow, 0)

    def _zorow(r, _):
        obuf[r, pl.ds(0, 16)] = jnp.zeros((16,), jnp.float32)
        return 0
    lax.fori_loop(0, IROW, _zorow, 0)

    def _zblk(k, _):
        b = sid + NSUB * k
        pltpu.sync_copy(pb0.at[pl.ds(0, 128)], acc_s.at[pl.ds(b * 128, 128)])
        pltpu.sync_copy(obuf, acc_c.at[pl.ds(b * 128, 128)])
        return 0
    lax.fori_loop(0, NBLK // NSUB, _zblk, 0)
    plsc.subcore_barrier()

    def _orow(r, _):
        obuf[r, pl.ds(0, 16)] = jnp.ones((16,), jnp.float32)
        return 0
    lax.fori_loop(0, IROW, _orow, 0)

    def _load_idx(s, j):
        ch = sid + j * NSUB
        pltpu.sync_copy(ei3.at[0, pl.ds(2 * ch, 2)], idxs[s].at[pl.ds(0, 2)])
        pltpu.sync_copy(ei3.at[1, pl.ds(2 * ch, 2)], idxs[s].at[pl.ds(2, 2)])
        for h in range(2):
            for g in range(IROW // 16):
                gsl = pl.ds(g * 16, 16)
                idxs[s][h, gsl] = 2 * idxs[s][h, gsl] + cid

    def _start_fetch(s, j):
        ch = sid + j * NSUB
        for h in range(2):
            pltpu.async_copy(p_hbm.at[idxs[s].at[h]],
                             pbs[s].at[pl.ds(h * IROW, IROW)], sgs[s])
        pltpu.async_copy(q_hbm.at[cid, pl.ds(ch * QROWS, QROWS)],
                         qbs[s], sqs[s])

    def _wait_fetch(s):
        for h in range(2):
            pltpu.make_async_copy(p_hbm.at[idxs[s].at[h]],
                                  pbs[s].at[pl.ds(h * IROW, IROW)], sgs[s]).wait()
        pltpu.make_async_copy(q_hbm.at[cid, pl.ds(0, QROWS)], qbs[s], sqs[s]).wait()

    def _start_scatter(s, j):
        for h in range(2):
            pltpu.async_copy(pbs[s].at[pl.ds(h * IROW, IROW)],
                             acc_s.at[idxs[s].at[2 + h]], sss[s], add=True)

        @pl.when(cid == s)
        def _():
            for h in range(2):
                pltpu.async_copy(obuf, acc_c.at[idxs[s].at[2 + h]],
                                 scs[s], add=True)

    def _wait_scatter(s):
        for h in range(2):
            pltpu.make_async_copy(pbs[s].at[pl.ds(h * IROW, IROW)],
                                  acc_s.at[idxs[s].at[2 + h]], sss[s]).wait()

        @pl.when(cid == s)
        def _():
            for h in range(2):
                pltpu.make_async_copy(obuf, acc_c.at[idxs[s].at[2 + h]],
                                      scs[s]).wait()

    def _compute(s):
        pb, qb = pbs[s], qbs[s]

        @plsc.parallel_loop(0, CHUNK, 1, unroll=16)
        def _crow(r):
            qrow = r >> 3
            qcol = (r & 7) * DH
            for cc in range(DH // 16):
                g = pl.ds(cc * 16, 16)
                pb[r, g] = jnp.maximum(
                    pb[r, g] + qb[qrow, pl.ds(qcol + cc * 16, 16)], 0.0)

    npairs = (nmy + 1) // 2

    _load_idx(0, 0)
    _start_fetch(0, 0)

    def _pair(k, _):
        for s in (0, 1):
            j = 2 * k + s
            nxt = 1 - s

            @pl.when(j + 1 < nmy)
            def _():
                @pl.when(j >= 1)
                def _():
                    _wait_scatter(nxt)
                _load_idx(nxt, j + 1)
                _start_fetch(nxt, j + 1)

            @pl.when(j < nmy)
            def _():
                _wait_fetch(s)
                _compute(s)
                _start_scatter(s, j)
        return 0
    lax.fori_loop(0, npairs, _pair, 0)

    _wait_scatter(0)
    _wait_scatter(1)
    plsc.subcore_barrier()

    def _dblk(k, _):
        b = sid + NSUB * k
        pltpu.sync_copy(acc_s.at[pl.ds(b * 128, 128)],
                        sums_out.at[cid, pl.ds(b * 128, 128)])

        pltpu.sync_copy(acc_c.at[pl.ds(b * 128, 128)],
                        cnts_out.at[cid, pl.ds(b * 128, 128)])
        return 0
    lax.fori_loop(0, NBLK // NSUB, _dblk, 0)


def kernel(nfeat, edge_index, edge_attr, W_edge, W_ne, b_ne, W_self, b_self):
    f32 = jnp.float32
    b_self2 = b_self.reshape(1, D_OUT)
    wn_bot_h = jnp.stack([W_ne[D_IN:, :DH], W_ne[D_IN:, DH:]])
    bn_h = jnp.stack([b_ne[:DH], b_ne[DH:]]).reshape(2, 1, DH)
    ea8 = edge_attr.reshape(E // PACK, PACK * D_EDGE)
    ei3 = edge_index.reshape(2, E // IROW, IROW)

    PB = 400
    P = pl.pallas_call(
        _p_body,
        grid=(N // PB,),
        in_specs=[
            pl.BlockSpec((PB, D_IN), lambda i: (i, 0)),
            pl.BlockSpec((D_IN + D_OUT, D_OUT), lambda i: (0, 0)),
        ],
        out_specs=pl.BlockSpec((PB, D_OUT), lambda i: (i, 0)),
        out_shape=jax.ShapeDtypeStruct((N, D_OUT), f32),
    )(nfeat, W_ne).reshape(2 * N, DH)

    QB = 2000
    Q = pl.pallas_call(
        _q_body,
        grid=(2, E // PACK // QB),
        in_specs=[
            pl.BlockSpec((QB, PACK * D_EDGE), lambda h, i: (i, 0)),
            pl.BlockSpec((D_EDGE, D_OUT), lambda h, i: (0, 0)),
            pl.BlockSpec((1, D_OUT, DH), lambda h, i: (h, 0, 0)),
            pl.BlockSpec((1, 1, DH), lambda h, i: (h, 0, 0)),
        ],
        out_specs=pl.BlockSpec((1, QB, QW), lambda h, i: (h, i, 0)),
        out_shape=jax.ShapeDtypeStruct((2, E // PACK, QW), f32),
    )(ea8, W_edge, wn_bot_h, bn_h)

    mesh = plsc.VectorSubcoreMesh(core_axis_name="c", subcore_axis_name="s")
    sc = functools.partial(
        pl.kernel,
        mesh=mesh,
        compiler_params=pltpu.CompilerParams(use_tc_tiling_on_sc=False),
        out_type=(
            jax.ShapeDtypeStruct((2, NPAD, DH), f32),
            jax.ShapeDtypeStruct((2, NPAD, 16), f32),
        ),
        scratch_types=[
            pltpu.VMEM((4, IROW), jnp.int32),
            pltpu.VMEM((4, IROW), jnp.int32),
            pltpu.VMEM((CHUNK, DH), f32),
            pltpu.VMEM((CHUNK, DH), f32),
            pltpu.VMEM((QROWS, QW), f32),
            pltpu.VMEM((QROWS, QW), f32),
            pltpu.VMEM((IROW, 16), f32),
            pltpu.VMEM_SHARED((NPAD, DH), f32),
            pltpu.VMEM_SHARED((NPAD, 16), f32),
            pltpu.SemaphoreType.DMA,
            pltpu.SemaphoreType.DMA,
            pltpu.SemaphoreType.DMA,
            pltpu.SemaphoreType.DMA,
            pltpu.SemaphoreType.DMA,
            pltpu.SemaphoreType.DMA,
            pltpu.SemaphoreType.DMA,
            pltpu.SemaphoreType.DMA,
        ],
    )(_sc_edge_kernel)
    sums, cnts = sc(ei3, P, Q)

    FB = 400
    out = pl.pallas_call(
        _fin_body,
        grid=(N // FB,),
        in_specs=[
            pl.BlockSpec((2, FB, DH), lambda i: (0, i, 0)),
            pl.BlockSpec((2, FB, 16), lambda i: (0, i, 0)),
            pl.BlockSpec((FB, D_IN), lambda i: (i, 0)),
            pl.BlockSpec((D_IN, D_OUT), lambda i: (0, 0)),
            pl.BlockSpec((1, D_OUT), lambda i: (0, 0)),
        ],
        out_specs=pl.BlockSpec((FB, D_OUT), lambda i: (i, 0)),
        out_shape=jax.ShapeDtypeStruct((N, D_OUT), f32),
    )(sums, cnts, nfeat, W_self, b_self2)
    return out
